# baseline (speedup 1.0000x reference)
import functools

import jax
import jax.numpy as jnp
from jax import lax
from jax.experimental import pallas as pl
from jax.experimental.pallas import tpu as pltpu
from jax.experimental.pallas import tpu_sc as plsc

NUM_CORES = 2
NUM_SUBCORES = 16
LANES = 16
NUM_WORKERS = NUM_CORES * NUM_SUBCORES

POS_OFFSET = 2
CHUNK = 16


def _make_sc_kernel(B, S, H, dtype):
    rows = B * S
    assert rows % NUM_WORKERS == 0
    rows_per_w = rows // NUM_WORKERS
    assert rows_per_w % CHUNK == 0
    n_chunks = rows_per_w // CHUNK
    assert S % rows_per_w == 0
    vecs_per_row = H // LANES

    mesh = plsc.VectorSubcoreMesh(
        core_axis_name="c", subcore_axis_name="s",
        num_cores=NUM_CORES, num_subcores=NUM_SUBCORES)

    @functools.partial(
        pl.kernel,
        out_type=jax.ShapeDtypeStruct((rows, H), dtype),
        mesh=mesh,
        scratch_types=[
            pltpu.VMEM((CHUNK,), jnp.int32),
            pltpu.VMEM((CHUNK, H), dtype),
            pltpu.VMEM((CHUNK + 8, H), dtype),
            pltpu.SemaphoreType.DMA,
        ],
    )
    def body(ids_hbm, tok_tbl_hbm, pos_tbl_hbm, out_hbm,
             idx_v, tok_v, pos_v, sem):
        cid = lax.axis_index("c")
        sid = lax.axis_index("s")
        wid = sid * NUM_CORES + cid
        base = wid * rows_per_w
        s_base = base % S

        def chunk_body(c, carry):
            r0 = base + c * CHUNK
            sp = s_base + c * CHUNK
            pltpu.sync_copy(ids_hbm.at[pl.ds(r0, CHUNK)], idx_v)
            gather = pltpu.async_copy(tok_tbl_hbm.at[idx_v], tok_v, sem)
            pltpu.sync_copy(pos_tbl_hbm.at[pl.ds(sp, CHUNK + 8)], pos_v)
            gather.wait()

            def add_row(r, carry2):
                for j in range(vecs_per_row):
                    sl = pl.ds(j * LANES, LANES)
                    tok_v[r, sl] = tok_v[r, sl] + pos_v[r + POS_OFFSET, sl]
                return carry2

            lax.fori_loop(0, CHUNK, add_row, 0)
            pltpu.sync_copy(tok_v, out_hbm.at[pl.ds(r0, CHUNK)])
            return carry

        lax.fori_loop(0, n_chunks, chunk_body, 0)

    return body


def kernel(token_ids, token_table, pos_table):
    B, S = token_ids.shape
    V, H = token_table.shape
    ids_flat = token_ids.reshape(-1).astype(jnp.int32)
    sc = _make_sc_kernel(B, S, H, token_table.dtype)
    out = sc(ids_flat, token_table, pos_table)
    return out.reshape(B, S, H)


# speedup vs baseline: 2.2131x; 2.2131x over previous
import functools

import jax
import jax.numpy as jnp
from jax import lax
from jax.experimental import pallas as pl
from jax.experimental.pallas import tpu as pltpu
from jax.experimental.pallas import tpu_sc as plsc

NUM_CORES = 2
NUM_SUBCORES = 16
LANES = 16
NUM_WORKERS = NUM_CORES * NUM_SUBCORES

POS_OFFSET = 2
CHUNK = 8


def _make_sc_kernel(B, S, H, dtype):
    assert S % NUM_WORKERS == 0
    s_per_w = S // NUM_WORKERS
    assert s_per_w % CHUNK == 0
    n_chunks = s_per_w // CHUNK
    vecs_per_row = H // LANES

    mesh = plsc.VectorSubcoreMesh(
        core_axis_name="c", subcore_axis_name="s",
        num_cores=NUM_CORES, num_subcores=NUM_SUBCORES)

    @functools.partial(
        pl.kernel,
        out_type=jax.ShapeDtypeStruct((B * S, H), dtype),
        mesh=mesh,
        scratch_types=[
            pltpu.VMEM((B, s_per_w), jnp.int32),
            [pltpu.VMEM((CHUNK, H), dtype) for _ in range(B)],
            pltpu.VMEM((CHUNK, H), dtype),
            pltpu.SemaphoreType.DMA,
            pltpu.SemaphoreType.DMA,
        ],
    )
    def body(ids_hbm, tok_tbl_hbm, pos_hbm, out_hbm,
             idx_v, tok_vs, pos_v, gsem, wsem):
        cid = lax.axis_index("c")
        sid = lax.axis_index("s")
        wid = sid * NUM_CORES + cid
        s0 = wid * s_per_w

        for b in range(B):
            pltpu.sync_copy(ids_hbm.at[b, pl.ds(s0, s_per_w)], idx_v.at[b])

        def fire_gathers(c):
            return [
                pltpu.async_copy(
                    tok_tbl_hbm.at[idx_v.at[b, pl.ds(c * CHUNK, CHUNK)]],
                    tok_vs[b], gsem)
                for b in range(B)
            ]

        pending_g = fire_gathers(0)
        pltpu.sync_copy(pos_hbm.at[pl.ds(s0, CHUNK)], pos_v)

        for c in range(n_chunks):
            for cp in pending_g:
                cp.wait()

            def add_vec(j, carry):
                sl = pl.ds(j * LANES, LANES)
                for r in range(CHUNK):
                    pv = pos_v[r, sl]
                    for b in range(B):
                        tok_vs[b][r, sl] = tok_vs[b][r, sl] + pv
                return carry

            lax.fori_loop(0, vecs_per_row, add_vec, 0)

            pending_w = [
                pltpu.async_copy(
                    tok_vs[b],
                    out_hbm.at[pl.ds(b * S + s0 + c * CHUNK, CHUNK)], wsem)
                for b in range(B)
            ]

            if c + 1 < n_chunks:
                pltpu.sync_copy(
                    pos_hbm.at[pl.ds(s0 + (c + 1) * CHUNK, CHUNK)], pos_v)
                for cp in pending_w:
                    cp.wait()
                pending_g = fire_gathers(c + 1)
            else:
                for cp in pending_w:
                    cp.wait()

    return body


def kernel(token_ids, token_table, pos_table):
    B, S = token_ids.shape
    V, H = token_table.shape
    ids_i32 = token_ids.astype(jnp.int32)
    pos_sliced = lax.slice(pos_table, (POS_OFFSET, 0), (POS_OFFSET + S, H))
    sc = _make_sc_kernel(B, S, H, token_table.dtype)
    out = sc(ids_i32, token_table, pos_sliced)
    return out.reshape(B, S, H)


# speedup vs baseline: 3.0395x; 1.3734x over previous
import functools

import jax
import jax.numpy as jnp
from jax import lax
from jax.experimental import pallas as pl
from jax.experimental.pallas import tpu as pltpu
from jax.experimental.pallas import tpu_sc as plsc

NUM_CORES = 2
NUM_SUBCORES = 16
LANES = 16
NUM_WORKERS = NUM_CORES * NUM_SUBCORES

POS_OFFSET = 2
CHUNK = 8
NHALF = 2


def _make_sc_kernel(B, S, H, dtype):
    assert S % NUM_WORKERS == 0
    s_per_w = S // NUM_WORKERS
    assert s_per_w % CHUNK == 0
    n_chunks = s_per_w // CHUNK
    HH = H // NHALF
    n_items = n_chunks * NHALF
    assert s_per_w % LANES == 0

    mesh = plsc.VectorSubcoreMesh(
        core_axis_name="c", subcore_axis_name="s",
        num_cores=NUM_CORES, num_subcores=NUM_SUBCORES)

    @functools.partial(
        pl.kernel,
        out_type=jax.ShapeDtypeStruct((B * S, H), dtype),
        mesh=mesh,
        scratch_types=[
            pltpu.VMEM((B, s_per_w), jnp.int32),
            pltpu.VMEM((s_per_w,), jnp.int32),
            [[pltpu.VMEM((CHUNK, HH), dtype) for _ in range(B)]
             for _ in range(2)],
            [pltpu.VMEM((CHUNK, H), dtype) for _ in range(2)],
            pltpu.SemaphoreType.DMA,
            pltpu.SemaphoreType.DMA,
            pltpu.SemaphoreType.DMA,
        ],
    )
    def body(ids_hbm, tok_tbl_hbm, pos_tbl_hbm, out_hbm,
             idx_v, pos_idx_v, tok_sets, pos_vs, gsem, wsem, psem):
        cid = lax.axis_index("c")
        sid = lax.axis_index("s")
        wid = sid * NUM_CORES + cid
        s0 = wid * s_per_w

        for b in range(B):
            pltpu.sync_copy(ids_hbm.at[b, pl.ds(s0, s_per_w)], idx_v.at[b])
        for t in range(s_per_w // LANES):
            pos_idx_v[pl.ds(t * LANES, LANES)] = (
                lax.iota(jnp.int32, LANES) + (s0 + POS_OFFSET + t * LANES))

        def fire_gathers(i):
            c, h = i // NHALF, i % NHALF
            return [
                pltpu.async_copy(
                    tok_tbl_hbm.at[idx_v.at[b, pl.ds(c * CHUNK, CHUNK)],
                                   pl.ds(h * HH, HH)],
                    tok_sets[i % 2][b], gsem)
                for b in range(B)
            ]

        def fire_pos(c):
            return pltpu.async_copy(
                pos_tbl_hbm.at[pos_idx_v.at[pl.ds(c * CHUNK, CHUNK)]],
                pos_vs[c % 2], psem)

        pend_g = {0: fire_gathers(0)}
        pend_p = {0: fire_pos(0)}
        pend_w = {}

        for i in range(n_items):
            c, h = i // NHALF, i % NHALF
            if i + 1 < n_items:
                if i - 1 >= 0:
                    for cp in pend_w[i - 1]:
                        cp.wait()
                pend_g[i + 1] = fire_gathers(i + 1)
                if h == 0 and c + 1 < n_chunks:
                    pend_p[c + 1] = fire_pos(c + 1)

            for cp in pend_g[i]:
                cp.wait()
            if h == 0:
                pend_p[c].wait()

            bufs = tok_sets[i % 2]
            pos_v = pos_vs[c % 2]

            def add_vec(j, carry, _bufs=bufs, _pos=pos_v, _h=h):
                for r in range(CHUNK):
                    sl = pl.ds(j * LANES, LANES)
                    pv = _pos[r, pl.ds(_h * HH + j * LANES, LANES)]
                    for b in range(B):
                        _bufs[b][r, sl] = _bufs[b][r, sl] + pv
                return carry

            lax.fori_loop(0, HH // LANES, add_vec, 0)

            pend_w[i] = [
                pltpu.async_copy(
                    bufs[b],
                    out_hbm.at[pl.ds(b * S + s0 + c * CHUNK, CHUNK),
                               pl.ds(h * HH, HH)], wsem)
                for b in range(B)
            ]

        for cp in pend_w[n_items - 2]:
            cp.wait()
        for cp in pend_w[n_items - 1]:
            cp.wait()

    return body


def kernel(token_ids, token_table, pos_table):
    B, S = token_ids.shape
    V, H = token_table.shape
    ids_i32 = token_ids.astype(jnp.int32)
    sc = _make_sc_kernel(B, S, H, token_table.dtype)
    out = sc(ids_i32, token_table, pos_table)
    return out.reshape(B, S, H)
